# baseline (speedup 1.0000x reference)
import jax
import jax.numpy as jnp
from jax.experimental import pallas as pl
from jax.experimental.pallas import tpu as pltpu
from jax.experimental.pallas import tpu_sc as plsc

W = 256
N_WORKERS = 32
LANES = 16
NCHUNK = W // LANES


def kernel(x, emb_weight):
    batch, hist = x.shape
    vocab, emb_dim = emb_weight.shape
    n_w = batch // W
    n_tasks = hist * n_w
    per_worker = n_tasks // N_WORKERS

    tw = emb_weight.reshape(vocab // 2, 2 * emb_dim)
    idx_t = x.T.astype(jnp.int32)

    mesh = plsc.VectorSubcoreMesh(
        core_axis_name="core", subcore_axis_name="subcore"
    )

    @pl.kernel(
        out_type=jax.ShapeDtypeStruct((hist, emb_dim, batch), jnp.float32),
        mesh=mesh,
        scratch_types=[
            pltpu.VMEM((2, W), jnp.int32),
            pltpu.VMEM((2, W // 128, 128), jnp.int32),
            pltpu.VMEM((2, W), jnp.int32),
            pltpu.VMEM((2, W // 128, 128, 2 * emb_dim), jnp.float32),
            pltpu.VMEM((2, emb_dim, W), jnp.float32),
            pltpu.SemaphoreType.DMA((2,)),
            pltpu.SemaphoreType.DMA((2,)),
            pltpu.SemaphoreType.DMA((2,)),
        ],
        compiler_params=pltpu.CompilerParams(
            use_tc_tiling_on_sc=True,
            needs_layout_passes=False,
            disable_bounds_checks=True
        ),
    )
    def gather_kernel(
        tw_hbm, i_hbm, o_hbm,
        idx_v, half_v, off_v, buf_v, out_v,
        isem, gsem, osem,
    ):
        nc = jax.lax.axis_size("core")
        wid = jax.lax.axis_index("subcore") * nc + jax.lax.axis_index("core")
        t0 = wid * per_worker

        def hw(t):
            h = t // n_w
            return h, t - h * n_w

        def idx_copy(t, b):
            h, w = hw(t)
            return pltpu.make_async_copy(
                i_hbm.at[h, pl.ds(w * W, W)], idx_v.at[b], isem.at[b]
            )

        def gather_copies(b):
            return [
                pltpu.make_async_copy(
                    tw_hbm.at[half_v.at[b, p]], buf_v.at[b, p], gsem.at[b]
                )
                for p in range(W // 128)
            ]

        def out_copy(t, b):
            h, w = hw(t)
            return pltpu.make_async_copy(
                out_v.at[b], o_hbm.at[h, :, pl.ds(w * W, W)], osem.at[b]
            )

        def prep(b):
            for c in range(NCHUNK):
                s = pl.ds(c * LANES, LANES)
                v = idx_v[b, s]
                half_v[b, c // 8, pl.ds((c % 8) * LANES, LANES)] = (
                    jax.lax.shift_right_logical(v, 1)
                )
                off_v[b, s] = (v & 1) * emb_dim

        jrows = [
            jax.lax.iota(jnp.int32, LANES) + ((c % 8) * LANES)
            for c in range(NCHUNK)
        ]

        def transpose(b):
            offs = [off_v[b, pl.ds(c * LANES, LANES)] for c in range(NCHUNK)]

            @plsc.parallel_loop(0, emb_dim, step=1, unroll=4)
            def _(d):
                vals = [
                    plsc.load_gather(
                        buf_v.at[b, c // 8],
                        [jrows[c], offs[c] + d],
                    )
                    for c in range(NCHUNK)
                ]
                for c in range(NCHUNK):
                    out_v[b, d, pl.ds(c * LANES, LANES)] = vals[c]

        idx_copy(t0, 0).start()
        idx_copy(t0, 0).wait()
        prep(0)
        for cp in gather_copies(0):
            cp.start()
        idx_copy(t0 + 1, 1).start()

        @pl.loop(0, per_worker, step=2)
        def _(i):
            for b in (0, 1):
                nb = 1 - b
                t = t0 + i + b

                @pl.when(i + b + 1 < per_worker)
                def _():
                    idx_copy(t + 1, nb).wait()
                    prep(nb)
                    for cp in gather_copies(nb):
                        cp.start()

                @pl.when(i + b + 2 < per_worker)
                def _():
                    idx_copy(t + 2, b).start()

                @pl.when(i + b >= 2)
                def _():
                    out_copy(t - 2, b).wait()

                for cp in gather_copies(b):
                    cp.wait()
                transpose(b)
                out_copy(t, b).start()

        out_copy(t0 + per_worker - 2, 0).wait()
        out_copy(t0 + per_worker - 1, 1).wait()

    out = gather_kernel(tw, idx_t)
    return out.transpose(2, 0, 1)


# speedup vs baseline: 1.0919x; 1.0919x over previous
import jax
import jax.numpy as jnp
from jax.experimental import pallas as pl
from jax.experimental.pallas import tpu as pltpu
from jax.experimental.pallas import tpu_sc as plsc

W = 256
N_WORKERS = 32
LANES = 16
NCHUNK = W // LANES


def kernel(x, emb_weight):
    batch, hist = x.shape
    vocab, emb_dim = emb_weight.shape
    n_w = batch // W
    n_tasks = hist * n_w
    per_worker = n_tasks // N_WORKERS

    tw = emb_weight.reshape(vocab // 2, 2 * emb_dim)
    idx_t = x.T.astype(jnp.int32)

    mesh = plsc.VectorSubcoreMesh(
        core_axis_name="core", subcore_axis_name="subcore"
    )

    @pl.kernel(
        out_type=jax.ShapeDtypeStruct((hist, emb_dim, batch), jnp.float32),
        mesh=mesh,
        scratch_types=[
            pltpu.VMEM((2, W), jnp.int32),
            pltpu.VMEM((2, W // 128, 128), jnp.int32),
            pltpu.VMEM((2, W), jnp.int32),
            pltpu.VMEM((2, W // 128, 128, 2 * emb_dim), jnp.float32),
            pltpu.VMEM((2, emb_dim, W + 17), jnp.float32),
            pltpu.SemaphoreType.DMA((2,)),
            pltpu.SemaphoreType.DMA((2,)),
            pltpu.SemaphoreType.DMA((2,)),
        ],
        compiler_params=pltpu.CompilerParams(
            use_tc_tiling_on_sc=True,
            needs_layout_passes=False,
            disable_bounds_checks=True
        ),
    )
    def gather_kernel(
        tw_hbm, i_hbm, o_hbm,
        idx_v, half_v, off_v, buf_v, out_v,
        isem, gsem, osem,
    ):
        nc = jax.lax.axis_size("core")
        wid = jax.lax.axis_index("subcore") * nc + jax.lax.axis_index("core")
        t0 = wid * per_worker

        def hw(t):
            h = t // n_w
            return h, t - h * n_w

        def idx_copy(t, b):
            h, w = hw(t)
            return pltpu.make_async_copy(
                i_hbm.at[h, pl.ds(w * W, W)], idx_v.at[b], isem.at[b]
            )

        def gather_copies(b):
            return [
                pltpu.make_async_copy(
                    tw_hbm.at[half_v.at[b, p]], buf_v.at[b, p], gsem.at[b]
                )
                for p in range(W // 128)
            ]

        def out_copy(t, b):
            h, w = hw(t)
            return pltpu.make_async_copy(
                out_v.at[b, :, pl.ds(0, W)],
                o_hbm.at[h, :, pl.ds(w * W, W)],
                osem.at[b],
            )

        def prep(b):
            for c in range(NCHUNK):
                v = idx_v[b, pl.ds(c * LANES, LANES)]
                half_v[b, c // 8, pl.ds((c % 8) * LANES, LANES)] = (
                    jax.lax.shift_right_logical(v, 1)
                )
                off_v[b, pl.ds(c * LANES, LANES)] = (v & 1) * emb_dim

        dchunks = [
            jax.lax.iota(jnp.int32, LANES) + (c * LANES)
            for c in range(emb_dim // LANES)
        ]

        def transpose(b):
            for p in range(W // 128):

                @plsc.parallel_loop(0, 128, step=LANES)
                def _(g):
                    ov = off_v[b, pl.ds(p * 128 + g, LANES)]
                    for l in range(LANES):
                        r = g + l
                        off = ov[l]
                        col = jax.lax.broadcast(p * 128 + r, (LANES,))
                        for c in range(emb_dim // LANES):
                            vals = buf_v[
                                b, p, r, pl.ds(off + c * LANES, LANES)
                            ]
                            plsc.store_scatter(
                                out_v.at[b], [dchunks[c], col], vals
                            )

        idx_copy(t0, 0).start()
        idx_copy(t0, 0).wait()
        prep(0)
        for cp in gather_copies(0):
            cp.start()
        idx_copy(t0 + 1, 1).start()

        @pl.loop(0, per_worker, step=2)
        def _(i):
            for b in (0, 1):
                nb = 1 - b
                t = t0 + i + b

                @pl.when(i + b + 1 < per_worker)
                def _():
                    idx_copy(t + 1, nb).wait()
                    prep(nb)
                    for cp in gather_copies(nb):
                        cp.start()

                @pl.when(i + b + 2 < per_worker)
                def _():
                    idx_copy(t + 2, b).start()

                @pl.when(i + b >= 2)
                def _():
                    out_copy(t - 2, b).wait()

                for cp in gather_copies(b):
                    cp.wait()
                transpose(b)
                out_copy(t, b).start()

        out_copy(t0 + per_worker - 2, 0).wait()
        out_copy(t0 + per_worker - 1, 1).wait()

    out = gather_kernel(tw, idx_t)
    return out.transpose(2, 0, 1)


# speedup vs baseline: 1.2202x; 1.1175x over previous
import jax
import jax.numpy as jnp
from jax.experimental import pallas as pl
from jax.experimental.pallas import tpu as pltpu
from jax.experimental.pallas import tpu_sc as plsc

WINDOW = 512


def kernel(x, emb_weight):
    batch, hist = x.shape
    _, emb_dim = emb_weight.shape
    n_idx = batch * hist
    idx = x.reshape(1, n_idx).astype(jnp.int32)

    mesh = plsc.VectorSubcoreMesh(
        core_axis_name="core", subcore_axis_name="subcore"
    )

    @pl.kernel(
        out_type=jax.ShapeDtypeStruct((n_idx, emb_dim), jnp.float32),
        mesh=mesh,
        compiler_params=pltpu.CompilerParams(use_tc_tiling_on_sc=False),
    )
    def gather_kernel(table_hbm, i_hbm, o_hbm):
        def body(i_vmem, o_vmem):
            pltpu.sync_copy(table_hbm.at[i_vmem.at[0]], o_vmem)

        pltpu.emit_pipeline(
            body,
            grid=(n_idx // WINDOW,),
            in_specs=[
                pl.BlockSpec((1, WINDOW), index_map=lambda i: (0, i))
            ],
            out_specs=[
                pl.BlockSpec((WINDOW, emb_dim), index_map=lambda i: (i, 0))
            ],
            core_axis_name=("core", "subcore"),
            dimension_semantics=(pltpu.PARALLEL,),
        )(i_hbm, o_hbm)

    out = gather_kernel(emb_weight, idx)
    return out.reshape(batch, hist, emb_dim)
